# baseline (speedup 1.0000x reference)
import functools

import jax
import jax.numpy as jnp
from jax import lax
from jax.experimental import pallas as pl
from jax.experimental.pallas import tpu as pltpu
from jax.experimental.pallas import tpu_sc as plsc

NC = 2
NS = 16
L = 16
W = NC * NS
KB = 128
NQ = 4
SBE = 2048


def _ecap(E):
    return ((E + KB + SBE - 1) // SBE) * SBE

_SC_PARAMS = pltpu.CompilerParams(needs_layout_passes=False)


def _cdiv(a, b):
    return (a + b - 1) // b


def _mesh():
    return plsc.VectorSubcoreMesh(core_axis_name="c", subcore_axis_name="s")


def _make_partition(N, E):
    RPT = _cdiv(_cdiv(N, W), 8) * 8
    HB = RPT // NQ
    CHK = 6400
    NCHK = E // CHK
    FLUSH = 2048
    OCAP = FLUSH + CHK + KB + 48
    TRASH = OCAP + KB
    ECAP = _ecap(E)
    out_t = (
        jax.ShapeDtypeStruct((W * NQ * ECAP,), jnp.int32),
        jax.ShapeDtypeStruct((W * NQ * ECAP,), jnp.int32),
        jax.ShapeDtypeStruct((W * NQ * KB,), jnp.int32),
    )
    scratch = (
        [pltpu.VMEM((CHK,), jnp.int32), pltpu.VMEM((CHK,), jnp.int32)]
        + [pltpu.VMEM((TRASH + L,), jnp.int32) for _ in range(2 * NQ)]
        + [pltpu.VMEM((KB,), jnp.int32)]
    )

    @functools.partial(pl.kernel, out_type=out_t, mesh=_mesh(),
                       scratch_types=scratch, compiler_params=_SC_PARAMS)
    def part(src_hbm, dst_hbm, esrc_hbm, eldst_hbm, ecnt_hbm,
             sin_v, din_v, *rest):
        souts = rest[0:NQ]
        douts = rest[NQ:2 * NQ]
        cv = rest[2 * NQ]
        c = lax.axis_index("c")
        s = lax.axis_index("s")
        w = c * NS + s
        bases = [pl.multiple_of((NQ * w + q) * ECAP, KB) for q in range(NQ)]
        lo = w * RPT
        lane = lax.iota(jnp.int32, L)

        def flush_one(q, total, cnt):
            off = pl.multiple_of(bases[q] + total, KB)
            pltpu.sync_copy(souts[q].at[pl.ds(0, FLUSH)],
                            esrc_hbm.at[pl.ds(off, FLUSH)])
            pltpu.sync_copy(douts[q].at[pl.ds(0, FLUSH)],
                            eldst_hbm.at[pl.ds(off, FLUSH)])
            rem = cnt - FLUSH

            def mv(i, carry2):
                sv = souts[q][pl.ds(FLUSH + i * L, L)]
                dv = douts[q][pl.ds(FLUSH + i * L, L)]
                souts[q][pl.ds(i * L, L)] = sv
                douts[q][pl.ds(i * L, L)] = dv
                return carry2

            lax.fori_loop(0, (rem + L - 1) // L, mv, jnp.int32(0))
            return (total + FLUSH, rem)

        def chunk_body(k, carry):
            carry = list(carry)
            for q in range(NQ):
                carry[2 * q], carry[2 * q + 1] = lax.cond(
                    carry[2 * q + 1] > OCAP - CHK,
                    lambda a, q=q: flush_one(q, *a),
                    lambda a: a, (carry[2 * q], carry[2 * q + 1]))

            coff = pl.multiple_of(k * CHK, 8)
            pltpu.sync_copy(src_hbm.at[pl.ds(coff, CHK)], sin_v)
            pltpu.sync_copy(dst_hbm.at[pl.ds(coff, CHK)], din_v)

            def scan_body(i, cnts):
                sv = sin_v[pl.ds(i * L, L)]
                dv = din_v[pl.ds(i * L, L)]
                dl = dv - lo
                out = []
                for q in range(NQ):
                    m = (dl >= q * HB) & (dl < (q + 1) * HB)
                    cs = plsc.cumsum(m.astype(jnp.int32))
                    pos = jnp.where(m, cnts[q] + cs - 1, TRASH + lane)
                    plsc.store_scatter(souts[q], [pos], sv)
                    plsc.store_scatter(douts[q], [pos], dv)
                    out.append(cnts[q] + cs[L - 1])
                return tuple(out)

            cnts = lax.fori_loop(0, CHK // L, scan_body,
                                 tuple(carry[2 * q + 1] for q in range(NQ)))
            for q in range(NQ):
                carry[2 * q + 1] = cnts[q]
            return tuple(carry)

        carry = lax.fori_loop(0, NCHK, chunk_body,
                              tuple(jnp.int32(0) for _ in range(2 * NQ)))

        zpad = jnp.zeros((L,), jnp.int32)
        tpad = jnp.full((L,), N, jnp.int32)
        for q in range(NQ):
            cnt = carry[2 * q + 1]
            for j in range(KB // L):
                souts[q][pl.ds(cnt + j * L, L)] = zpad
                douts[q][pl.ds(cnt + j * L, L)] = tpad

        for q in range(NQ):
            total, cnt = carry[2 * q], carry[2 * q + 1]
            nf = (cnt + KB - 1) // KB

            def final_flush(j, carry2, q=q, total=total):
                off = pl.multiple_of(bases[q] + total + j * KB, KB)
                pltpu.sync_copy(souts[q].at[pl.ds(j * KB, KB)],
                                esrc_hbm.at[pl.ds(off, KB)])
                pltpu.sync_copy(douts[q].at[pl.ds(j * KB, KB)],
                                eldst_hbm.at[pl.ds(off, KB)])
                return carry2

            lax.fori_loop(0, nf, final_flush, jnp.int32(0))
            for j in range(KB // L):
                cv[pl.ds(j * L, L)] = zpad + (total + cnt)
            pltpu.sync_copy(
                cv, ecnt_hbm.at[pl.ds(
                    pl.multiple_of((NQ * w + q) * KB, KB), KB)])

    return part


def _make_agg(N, Dc, E):
    RPT = _cdiv(_cdiv(N, W), 8) * 8
    HB = RPT // NQ
    TR = HB + 8
    KBA = 16
    SBB = SBE // KBA
    NP = W * RPT
    ECAP = _ecap(E)
    scratch = [
        pltpu.VMEM((SBE,), jnp.int32), pltpu.VMEM((SBE,), jnp.int32),
        pltpu.VMEM((KB,), jnp.int32),
        pltpu.VMEM((4, KBA, Dc), jnp.float32),
        pltpu.VMEM((TR, Dc), jnp.float32),
        pltpu.SemaphoreType.DMA,
        pltpu.SemaphoreType.DMA,
        pltpu.SemaphoreType.DMA,
        pltpu.SemaphoreType.DMA,
    ]

    @functools.partial(
        pl.kernel, out_type=jax.ShapeDtypeStruct((NP, Dc), jnp.float32),
        mesh=_mesh(), scratch_types=scratch)
    def agg(h_hbm, z_hbm, esrc_hbm, eldst_hbm, ecnt_hbm, agg_hbm,
            sbs_v, sbd_v, cv, rows_v, acc_v, sem0, sem1, sem2, sem3):
        c = lax.axis_index("c")
        s = lax.axis_index("s")
        w = c * NS + s
        sems = (sem0, sem1, sem2, sem3)
        NBUF = 4

        def gather(j_local, b, sems=sems):
            ioff = pl.multiple_of(j_local * KBA, KBA)
            return pltpu.make_async_copy(
                h_hbm.at[sbs_v.at[pl.ds(ioff, KBA)]],
                rows_v.at[b], sems[b])

        def q_body(q, qcarry):
            lbase = pl.multiple_of((NQ * w + q) * ECAP, KBA)
            brow = pl.multiple_of((NQ * w + q) * HB, 8)
            pltpu.sync_copy(z_hbm, acc_v)
            pltpu.sync_copy(
                ecnt_hbm.at[pl.ds(pl.multiple_of((NQ * w + q) * KB, KB), KB)],
                cv)
            cnt = cv[pl.ds(0, L)][0]
            nb = (cnt + (KBA - 1)) // KBA
            nsb = (nb + (SBB - 1)) // SBB

            def super_body(t, carry, lbase=lbase, brow=brow, nb=nb):
                soff = pl.multiple_of(lbase + t * SBE, KBA)
                pltpu.sync_copy(esrc_hbm.at[pl.ds(soff, SBE)], sbs_v)
                pltpu.sync_copy(eldst_hbm.at[pl.ds(soff, SBE)], sbd_v)
                jmax = jnp.minimum(nb - t * SBB, SBB)

                for p in range(3):
                    @pl.when(p < jmax)
                    def _prime(p=p):
                        gather(p, p).start()

                def pair_body(jj, carry2, brow=brow, jmax=jmax):
                    for b in range(NBUF):
                        j = jj * NBUF + b
                        nxt = (b + NBUF - 1) % NBUF

                        @pl.when(j + NBUF - 1 < jmax)
                        def _start_next(j=j, nxt=nxt):
                            gather(j + NBUF - 1, nxt).start()

                        @pl.when(j < jmax)
                        def _accum(j=j, b=b, brow=brow):
                            gather(j, b).wait()
                            rlocs = []
                            for e in range(KBA // L):
                                dvec = sbd_v[pl.ds(j * KBA + e * L, L)]
                                rl = jnp.clip(dvec - brow, 0, HB)
                                for lane in range(L):
                                    rlocs.append(rl[lane])

                            def kbody(k, carry3):
                                for u in range(2):
                                    off2 = pl.multiple_of(k * 2 * L + u * L,
                                                          L)
                                    for ei, rloc in enumerate(rlocs):
                                        plsc.addupdate(
                                            acc_v.at[rloc, pl.ds(off2, L)],
                                            rows_v[b, ei, pl.ds(off2, L)])
                                return carry3

                            lax.fori_loop(0, Dc // (2 * L), kbody,
                                          jnp.int32(0))
                    return carry2

                lax.fori_loop(0, (jmax + NBUF - 1) // NBUF, pair_body,
                              jnp.int32(0))
                return carry

            lax.fori_loop(0, nsb, super_body, jnp.int32(0))
            pltpu.sync_copy(acc_v.at[pl.ds(0, HB)],
                            agg_hbm.at[pl.ds(brow, HB)])
            return qcarry

        lax.fori_loop(0, NQ, q_body, jnp.int32(0))

    return agg


def _make_head(N, B):
    BPW = B // W

    @functools.partial(
        pl.kernel, out_type=jax.ShapeDtypeStruct((B,), jnp.float32),
        mesh=_mesh(), compiler_params=_SC_PARAMS,
        scratch_types=[pltpu.VMEM((N,), jnp.float32),
                       pltpu.VMEM((BPW,), jnp.int32),
                       pltpu.VMEM((BPW,), jnp.float32)])
    def head(scores_hbm, batch_hbm, out_hbm, sc_v, idx_v, o_v):
        c = lax.axis_index("c")
        s = lax.axis_index("s")
        w = c * NS + s
        boff = pl.multiple_of(w * BPW, BPW)
        pltpu.sync_copy(scores_hbm, sc_v)
        pltpu.sync_copy(batch_hbm.at[pl.ds(boff, BPW)], idx_v)
        for k in range(BPW // L):
            idx16 = idx_v[pl.ds(k * L, L)]
            o_v[pl.ds(k * L, L)] = plsc.load_gather(sc_v, [idx16])
        pltpu.sync_copy(o_v, out_hbm.at[pl.ds(boff, BPW)])

    return head


def _make_tc(N, D, H, head):
    RB = 1000

    def body(cnt_b, agg_b, h_b, wl_b, wr_b, bl_b, *rest):
        if head:
            linw_b, linb_b, out_b, sc_b = rest
        else:
            (out_b,) = rest
        inv = 1.0 / jnp.maximum(cnt_b[:, 0:1], 1.0)
        acc = jnp.dot(agg_b[...] * inv, wl_b[...],
                      preferred_element_type=jnp.float32)
        acc = acc + jnp.dot(h_b[...], wr_b[...],
                            preferred_element_type=jnp.float32)
        hn = jnp.maximum(acc + bl_b[...], 0.0)
        out_b[...] = hn
        if head:
            sc_b[...] = jnp.dot(hn, linw_b[...],
                                preferred_element_type=jnp.float32) + linb_b[...]

    in_specs = [
        pl.BlockSpec((RB, L), lambda i: (i, 0)),
        pl.BlockSpec((RB, D), lambda i: (i, 0)),
        pl.BlockSpec((RB, D), lambda i: (i, 0)),
        pl.BlockSpec((D, H), lambda i: (0, 0)),
        pl.BlockSpec((D, H), lambda i: (0, 0)),
        pl.BlockSpec((1, H), lambda i: (0, 0)),
    ]
    out_specs = pl.BlockSpec((RB, H), lambda i: (i, 0))
    out_shape = jax.ShapeDtypeStruct((N, H), jnp.float32)
    if head:
        in_specs += [pl.BlockSpec((H, 1), lambda i: (0, 0)),
                     pl.BlockSpec((1, 1), lambda i: (0, 0))]
        out_specs = (out_specs, pl.BlockSpec((RB, 1), lambda i: (i, 0)))
        out_shape = (out_shape, jax.ShapeDtypeStruct((N, 1), jnp.float32))
    return pl.pallas_call(
        body, grid=(N // RB,),
        in_specs=in_specs, out_specs=out_specs, out_shape=out_shape,
        compiler_params=pltpu.CompilerParams(
            dimension_semantics=("parallel",)))


def kernel(x, edge_index, batch, Wl1, bl1, Wr1, Wl2, bl2, Wr2,
           Wl3, bl3, Wr3, linW, linb):
    N, D = x.shape
    H = Wl1.shape[0]
    E = edge_index.shape[1]
    B = batch.shape[0]

    part = _make_partition(N, E)
    esrc, eldst, ecnt = part(edge_index[0], edge_index[1])

    RPT = _cdiv(_cdiv(N, W), 8) * 8
    TR = RPT // NQ + 8
    zDa = jnp.zeros((TR, D + KB), jnp.float32)
    zH = jnp.zeros((TR, H), jnp.float32)

    xa = jnp.concatenate([x, jnp.ones((N, KB), jnp.float32)], axis=1)
    agg1w = _make_agg(N, D + KB, E)(xa, zDa, esrc, eldst, ecnt)
    agg1 = agg1w[:, :D]
    cnt = agg1w[:, D:D + L]
    h1 = _make_tc(N, D, H, False)(cnt, agg1, x, Wl1.T, Wr1.T,
                                  bl1.reshape(1, H))

    aggf = _make_agg(N, H, E)
    a2 = aggf(h1, zH, esrc, eldst, ecnt)
    h2 = _make_tc(N, H, H, False)(cnt, a2, h1, Wl2.T, Wr2.T,
                                  bl2.reshape(1, H))

    a3 = aggf(h2, zH, esrc, eldst, ecnt)
    h3, scores = _make_tc(N, H, H, True)(
        cnt, a3, h2, Wl3.T, Wr3.T, bl3.reshape(1, H),
        linW.T, linb.reshape(1, 1))

    return _make_head(N, B)(scores.reshape(N), batch)


# speedup vs baseline: 1.0364x; 1.0364x over previous
import functools

import jax
import jax.numpy as jnp
from jax import lax
from jax.experimental import pallas as pl
from jax.experimental.pallas import tpu as pltpu
from jax.experimental.pallas import tpu_sc as plsc

NC = 2
NS = 16
L = 16
W = NC * NS
KB = 128
NQ = 4
SBE = 2048


def _ecap(E):
    return ((E + KB + SBE - 1) // SBE) * SBE

_SC_PARAMS = pltpu.CompilerParams(needs_layout_passes=False)


def _cdiv(a, b):
    return (a + b - 1) // b


def _mesh():
    return plsc.VectorSubcoreMesh(core_axis_name="c", subcore_axis_name="s")


def _make_partition(N, E):
    RPT = _cdiv(_cdiv(N, W), 8) * 8
    HB = RPT // NQ
    CHK = 6400
    NCHK = E // CHK
    FLUSH = 2048
    OCAP = FLUSH + CHK + KB + 48
    TRASH = OCAP + KB
    ECAP = _ecap(E)
    out_t = (
        jax.ShapeDtypeStruct((W * NQ * ECAP,), jnp.int32),
        jax.ShapeDtypeStruct((W * NQ * ECAP,), jnp.int32),
        jax.ShapeDtypeStruct((W * NQ * KB,), jnp.int32),
    )
    scratch = (
        [pltpu.VMEM((CHK,), jnp.int32), pltpu.VMEM((CHK,), jnp.int32)]
        + [pltpu.VMEM((TRASH + L,), jnp.int32) for _ in range(2 * NQ)]
        + [pltpu.VMEM((KB,), jnp.int32)]
    )

    @functools.partial(pl.kernel, out_type=out_t, mesh=_mesh(),
                       scratch_types=scratch, compiler_params=_SC_PARAMS)
    def part(src_hbm, dst_hbm, esrc_hbm, eldst_hbm, ecnt_hbm,
             sin_v, din_v, *rest):
        souts = rest[0:NQ]
        douts = rest[NQ:2 * NQ]
        cv = rest[2 * NQ]
        c = lax.axis_index("c")
        s = lax.axis_index("s")
        w = c * NS + s
        bases = [pl.multiple_of((NQ * w + q) * ECAP, KB) for q in range(NQ)]
        lo = w * RPT
        lane = lax.iota(jnp.int32, L)

        def flush_one(q, total, cnt):
            off = pl.multiple_of(bases[q] + total, KB)
            pltpu.sync_copy(souts[q].at[pl.ds(0, FLUSH)],
                            esrc_hbm.at[pl.ds(off, FLUSH)])
            pltpu.sync_copy(douts[q].at[pl.ds(0, FLUSH)],
                            eldst_hbm.at[pl.ds(off, FLUSH)])
            rem = cnt - FLUSH

            def mv(i, carry2):
                sv = souts[q][pl.ds(FLUSH + i * L, L)]
                dv = douts[q][pl.ds(FLUSH + i * L, L)]
                souts[q][pl.ds(i * L, L)] = sv
                douts[q][pl.ds(i * L, L)] = dv
                return carry2

            lax.fori_loop(0, (rem + L - 1) // L, mv, jnp.int32(0))
            return (total + FLUSH, rem)

        def chunk_body(k, carry):
            carry = list(carry)
            for q in range(NQ):
                carry[2 * q], carry[2 * q + 1] = lax.cond(
                    carry[2 * q + 1] > OCAP - CHK,
                    lambda a, q=q: flush_one(q, *a),
                    lambda a: a, (carry[2 * q], carry[2 * q + 1]))

            coff = pl.multiple_of(k * CHK, 8)
            pltpu.sync_copy(src_hbm.at[pl.ds(coff, CHK)], sin_v)
            pltpu.sync_copy(dst_hbm.at[pl.ds(coff, CHK)], din_v)

            def scan_body(i, cnts):
                sv = sin_v[pl.ds(i * L, L)]
                dv = din_v[pl.ds(i * L, L)]
                dl = dv - lo
                out = []
                for q in range(NQ):
                    m = (dl >= q * HB) & (dl < (q + 1) * HB)
                    cs = plsc.cumsum(m.astype(jnp.int32))
                    pos = jnp.where(m, cnts[q] + cs - 1, TRASH + lane)
                    plsc.store_scatter(souts[q], [pos], sv)
                    plsc.store_scatter(douts[q], [pos], dv)
                    out.append(cnts[q] + cs[L - 1])
                return tuple(out)

            cnts = lax.fori_loop(0, CHK // L, scan_body,
                                 tuple(carry[2 * q + 1] for q in range(NQ)))
            for q in range(NQ):
                carry[2 * q + 1] = cnts[q]
            return tuple(carry)

        carry = lax.fori_loop(0, NCHK, chunk_body,
                              tuple(jnp.int32(0) for _ in range(2 * NQ)))

        zpad = jnp.zeros((L,), jnp.int32)
        tpad = jnp.full((L,), N, jnp.int32)
        for q in range(NQ):
            cnt = carry[2 * q + 1]
            for j in range(KB // L):
                souts[q][pl.ds(cnt + j * L, L)] = zpad
                douts[q][pl.ds(cnt + j * L, L)] = tpad

        for q in range(NQ):
            total, cnt = carry[2 * q], carry[2 * q + 1]
            nf = (cnt + KB - 1) // KB

            def final_flush(j, carry2, q=q, total=total):
                off = pl.multiple_of(bases[q] + total + j * KB, KB)
                pltpu.sync_copy(souts[q].at[pl.ds(j * KB, KB)],
                                esrc_hbm.at[pl.ds(off, KB)])
                pltpu.sync_copy(douts[q].at[pl.ds(j * KB, KB)],
                                eldst_hbm.at[pl.ds(off, KB)])
                return carry2

            lax.fori_loop(0, nf, final_flush, jnp.int32(0))
            for j in range(KB // L):
                cv[pl.ds(j * L, L)] = zpad + (total + cnt)
            pltpu.sync_copy(
                cv, ecnt_hbm.at[pl.ds(
                    pl.multiple_of((NQ * w + q) * KB, KB), KB)])

    return part


def _make_agg(N, Dc, E):
    RPT = _cdiv(_cdiv(N, W), 8) * 8
    HB = RPT // NQ
    TR = HB + 8
    KBA = 16
    SBB = SBE // KBA
    NP = W * RPT
    ECAP = _ecap(E)
    scratch = [
        pltpu.VMEM((SBE,), jnp.int32), pltpu.VMEM((SBE,), jnp.int32),
        pltpu.VMEM((KB,), jnp.int32),
        pltpu.VMEM((4, KBA, Dc), jnp.float32),
        pltpu.VMEM((TR, Dc), jnp.float32),
        pltpu.SemaphoreType.DMA,
        pltpu.SemaphoreType.DMA,
        pltpu.SemaphoreType.DMA,
        pltpu.SemaphoreType.DMA,
    ]

    @functools.partial(
        pl.kernel, out_type=jax.ShapeDtypeStruct((NP, Dc), jnp.float32),
        mesh=_mesh(), scratch_types=scratch)
    def agg(h_hbm, z_hbm, esrc_hbm, eldst_hbm, ecnt_hbm, agg_hbm,
            sbs_v, sbd_v, cv, rows_v, acc_v, sem0, sem1, sem2, sem3):
        c = lax.axis_index("c")
        s = lax.axis_index("s")
        w = c * NS + s
        sems = (sem0, sem1, sem2, sem3)
        NBUF = 4

        def gather(j_local, b, sems=sems):
            ioff = pl.multiple_of(j_local * KBA, KBA)
            return pltpu.make_async_copy(
                h_hbm.at[sbs_v.at[pl.ds(ioff, KBA)]],
                rows_v.at[b], sems[b])

        def q_body(q, qcarry):
            lbase = pl.multiple_of((NQ * w + q) * ECAP, KBA)
            brow = pl.multiple_of((NQ * w + q) * HB, 8)
            pltpu.sync_copy(z_hbm, acc_v)
            pltpu.sync_copy(
                ecnt_hbm.at[pl.ds(pl.multiple_of((NQ * w + q) * KB, KB), KB)],
                cv)
            cnt = cv[pl.ds(0, L)][0]
            nb = (cnt + (KBA - 1)) // KBA
            nsb = (nb + (SBB - 1)) // SBB

            def super_body(t, carry, lbase=lbase, brow=brow, nb=nb):
                soff = pl.multiple_of(lbase + t * SBE, KBA)
                pltpu.sync_copy(esrc_hbm.at[pl.ds(soff, SBE)], sbs_v)
                pltpu.sync_copy(eldst_hbm.at[pl.ds(soff, SBE)], sbd_v)
                jmax = jnp.minimum(nb - t * SBB, SBB)

                for p in range(3):
                    @pl.when(p < jmax)
                    def _prime(p=p):
                        gather(p, p).start()

                def pair_body(jj, carry2, brow=brow, jmax=jmax):
                    for b in range(NBUF):
                        j = jj * NBUF + b
                        nxt = (b + NBUF - 1) % NBUF

                        @pl.when(j + NBUF - 1 < jmax)
                        def _start_next(j=j, nxt=nxt):
                            gather(j + NBUF - 1, nxt).start()

                        @pl.when(j < jmax)
                        def _accum(j=j, b=b, brow=brow):
                            gather(j, b).wait()
                            rlocs = []
                            for e in range(KBA // L):
                                dvec = sbd_v[pl.ds(j * KBA + e * L, L)]
                                rl = jnp.clip(dvec - brow, 0, HB)
                                for lane in range(L):
                                    rlocs.append(rl[lane])

                            def kbody(k, carry3):
                                off2 = pl.multiple_of(k * L, L)
                                for ei, rloc in enumerate(rlocs):
                                    plsc.addupdate(
                                        acc_v.at[rloc, pl.ds(off2, L)],
                                        rows_v[b, ei, pl.ds(off2, L)])
                                return carry3

                            lax.fori_loop(0, Dc // L, kbody, jnp.int32(0))
                    return carry2

                lax.fori_loop(0, (jmax + NBUF - 1) // NBUF, pair_body,
                              jnp.int32(0))
                return carry

            lax.fori_loop(0, nsb, super_body, jnp.int32(0))
            pltpu.sync_copy(acc_v.at[pl.ds(0, HB)],
                            agg_hbm.at[pl.ds(brow, HB)])
            return qcarry

        lax.fori_loop(0, NQ, q_body, jnp.int32(0))

    return agg


def _make_head(N, B):
    BPW = B // W

    @functools.partial(
        pl.kernel, out_type=jax.ShapeDtypeStruct((B,), jnp.float32),
        mesh=_mesh(), compiler_params=_SC_PARAMS,
        scratch_types=[pltpu.VMEM((N,), jnp.float32),
                       pltpu.VMEM((BPW,), jnp.int32),
                       pltpu.VMEM((BPW,), jnp.float32)])
    def head(scores_hbm, batch_hbm, out_hbm, sc_v, idx_v, o_v):
        c = lax.axis_index("c")
        s = lax.axis_index("s")
        w = c * NS + s
        boff = pl.multiple_of(w * BPW, BPW)
        pltpu.sync_copy(scores_hbm, sc_v)
        pltpu.sync_copy(batch_hbm.at[pl.ds(boff, BPW)], idx_v)
        for k in range(BPW // L):
            idx16 = idx_v[pl.ds(k * L, L)]
            o_v[pl.ds(k * L, L)] = plsc.load_gather(sc_v, [idx16])
        pltpu.sync_copy(o_v, out_hbm.at[pl.ds(boff, BPW)])

    return head


def _make_tc(N, D, H, head):
    RB = 1000

    def body(cnt_b, agg_b, h_b, wl_b, wr_b, bl_b, *rest):
        if head:
            linw_b, linb_b, out_b, sc_b = rest
        else:
            (out_b,) = rest
        inv = 1.0 / jnp.maximum(cnt_b[:, 0:1], 1.0)
        acc = jnp.dot(agg_b[...] * inv, wl_b[...],
                      preferred_element_type=jnp.float32)
        acc = acc + jnp.dot(h_b[...], wr_b[...],
                            preferred_element_type=jnp.float32)
        hn = jnp.maximum(acc + bl_b[...], 0.0)
        out_b[...] = hn
        if head:
            sc_b[...] = jnp.dot(hn, linw_b[...],
                                preferred_element_type=jnp.float32) + linb_b[...]

    in_specs = [
        pl.BlockSpec((RB, L), lambda i: (i, 0)),
        pl.BlockSpec((RB, D), lambda i: (i, 0)),
        pl.BlockSpec((RB, D), lambda i: (i, 0)),
        pl.BlockSpec((D, H), lambda i: (0, 0)),
        pl.BlockSpec((D, H), lambda i: (0, 0)),
        pl.BlockSpec((1, H), lambda i: (0, 0)),
    ]
    out_specs = pl.BlockSpec((RB, H), lambda i: (i, 0))
    out_shape = jax.ShapeDtypeStruct((N, H), jnp.float32)
    if head:
        in_specs += [pl.BlockSpec((H, 1), lambda i: (0, 0)),
                     pl.BlockSpec((1, 1), lambda i: (0, 0))]
        out_specs = (out_specs, pl.BlockSpec((RB, 1), lambda i: (i, 0)))
        out_shape = (out_shape, jax.ShapeDtypeStruct((N, 1), jnp.float32))
    return pl.pallas_call(
        body, grid=(N // RB,),
        in_specs=in_specs, out_specs=out_specs, out_shape=out_shape,
        compiler_params=pltpu.CompilerParams(
            dimension_semantics=("parallel",)))


def kernel(x, edge_index, batch, Wl1, bl1, Wr1, Wl2, bl2, Wr2,
           Wl3, bl3, Wr3, linW, linb):
    N, D = x.shape
    H = Wl1.shape[0]
    E = edge_index.shape[1]
    B = batch.shape[0]

    part = _make_partition(N, E)
    esrc, eldst, ecnt = part(edge_index[0], edge_index[1])

    RPT = _cdiv(_cdiv(N, W), 8) * 8
    TR = RPT // NQ + 8
    zDa = jnp.zeros((TR, D + KB), jnp.float32)
    zH = jnp.zeros((TR, H), jnp.float32)

    xa = jnp.concatenate([x, jnp.ones((N, KB), jnp.float32)], axis=1)
    agg1w = _make_agg(N, D + KB, E)(xa, zDa, esrc, eldst, ecnt)
    agg1 = agg1w[:, :D]
    cnt = agg1w[:, D:D + L]
    h1 = _make_tc(N, D, H, False)(cnt, agg1, x, Wl1.T, Wr1.T,
                                  bl1.reshape(1, H))

    aggf = _make_agg(N, H, E)
    a2 = aggf(h1, zH, esrc, eldst, ecnt)
    h2 = _make_tc(N, H, H, False)(cnt, a2, h1, Wl2.T, Wr2.T,
                                  bl2.reshape(1, H))

    a3 = aggf(h2, zH, esrc, eldst, ecnt)
    h3, scores = _make_tc(N, H, H, True)(
        cnt, a3, h2, Wl3.T, Wr3.T, bl3.reshape(1, H),
        linW.T, linb.reshape(1, 1))

    return _make_head(N, B)(scores.reshape(N), batch)
